# baseline (speedup 1.0000x reference)
import jax
import jax.numpy as jnp
from jax.experimental import pallas as pl
from jax.experimental.pallas import tpu as pltpu

_BN_EPS = 1e-5
_VMEM_LIMIT = 64 * 1024 * 1024


def _make_conv_stats_kernel(h, w, cin, cout, kh_taps, kw_taps):
    hw = h * w

    def _conv_stats_kernel(x_ref, w_ref, ct_ref, s_ref, q_ref):
        kp = w_ref.shape[1]
        xt = x_ref[0].reshape(cin, hw).T

        zrow = jnp.zeros((w, cin), jnp.float32)
        xc = jnp.concatenate([
            jnp.concatenate([zrow, xt[:-w]], axis=0),
            xt,
            jnp.concatenate([xt[w:], zrow], axis=0),
            jnp.zeros((hw, kp - kh_taps * cin), jnp.float32),
        ], axis=-1).astype(jnp.bfloat16)

        pix = jax.lax.broadcasted_iota(jnp.int32, (hw, 1), 0)
        col = pix % w
        z1 = jnp.zeros((1, kp), jnp.bfloat16)
        xm = jnp.where(col == 0, jnp.bfloat16(0),
                       jnp.concatenate([z1, xc[:-1]], axis=0))
        xp = jnp.where(col == w - 1, jnp.bfloat16(0),
                       jnp.concatenate([xc[1:], z1], axis=0))

        big = jnp.concatenate([xm, xc, xp], axis=-1)
        acc = jax.lax.dot_general(
            big, w_ref[...].reshape(kw_taps * kp, cout),
            dimension_numbers=(((1,), (0,)), ((), ())),
            preferred_element_type=jnp.float32)

        s_ref[...] = jnp.sum(acc, axis=0).reshape(1, 1, cout)
        q_ref[...] = jnp.sum(acc * acc, axis=0).reshape(1, 1, cout)
        ct_ref[...] = acc.T.astype(jnp.bfloat16).reshape(1, cout, hw)

    return _conv_stats_kernel


def _bn_relu_kernel(ct_ref, sc_ref, sh_ref, o_ref):
    v = ct_ref[0].astype(jnp.float32)
    y = jnp.maximum(v * sc_ref[...] + sh_ref[...], 0.0)
    o_ref[...] = y.reshape(o_ref.shape)


def kernel(x_nchw, w_oihw, bias, gamma, beta):
    del bias
    n, cin, h, w = x_nchw.shape
    cout, _, kh, kw = w_oihw.shape
    hw = h * w
    kp = 4 * cin

    w3 = jnp.transpose(w_oihw, (3, 2, 1, 0)).reshape(kw, kh * cin, cout)
    w3 = jnp.pad(w3, ((0, 0), (0, kp - kh * cin), (0, 0))).astype(jnp.bfloat16)

    conv_flops = 2 * n * hw * kh * kw * cin * cout
    conv_bytes = 4 * x_nchw.size + 2 * (w3.size + n * hw * cout)

    convt, csum, csq = pl.pallas_call(
        _make_conv_stats_kernel(h, w, cin, cout, kh, kw),
        grid=(n,),
        in_specs=[
            pl.BlockSpec((1, cin, h, w), lambda i: (i, 0, 0, 0)),
            pl.BlockSpec((kw, kp, cout), lambda i: (0, 0, 0)),
        ],
        out_specs=[
            pl.BlockSpec((1, cout, hw), lambda i: (i, 0, 0)),
            pl.BlockSpec((1, 1, cout), lambda i: (i, 0, 0)),
            pl.BlockSpec((1, 1, cout), lambda i: (i, 0, 0)),
        ],
        out_shape=(
            jax.ShapeDtypeStruct((n, cout, hw), jnp.bfloat16),
            jax.ShapeDtypeStruct((n, 1, cout), jnp.float32),
            jax.ShapeDtypeStruct((n, 1, cout), jnp.float32),
        ),
        compiler_params=pltpu.CompilerParams(
            dimension_semantics=("parallel",),
            vmem_limit_bytes=_VMEM_LIMIT),
        cost_estimate=pl.CostEstimate(
            flops=conv_flops, transcendentals=0, bytes_accessed=conv_bytes),
    )(x_nchw, w3)

    cnt = float(n * hw)
    mean = csum.sum(axis=(0, 1)) / cnt
    var = jnp.maximum(csq.sum(axis=(0, 1)) / cnt - mean * mean, 0.0)
    scale = gamma.astype(jnp.float32) * jax.lax.rsqrt(var + _BN_EPS)
    shift = beta.astype(jnp.float32) - mean * scale

    out = pl.pallas_call(
        _bn_relu_kernel,
        grid=(n,),
        in_specs=[
            pl.BlockSpec((1, cout, hw), lambda i: (i, 0, 0)),
            pl.BlockSpec((cout, 1), lambda i: (0, 0)),
            pl.BlockSpec((cout, 1), lambda i: (0, 0)),
        ],
        out_specs=pl.BlockSpec((1, cout, h, w), lambda i: (i, 0, 0, 0)),
        out_shape=jax.ShapeDtypeStruct((n, cout, h, w), jnp.float32),
        compiler_params=pltpu.CompilerParams(
            dimension_semantics=("parallel",),
            vmem_limit_bytes=_VMEM_LIMIT),
        cost_estimate=pl.CostEstimate(
            flops=3 * n * hw * cout, transcendentals=0,
            bytes_accessed=6 * n * hw * cout),
    )(convt, scale.reshape(cout, 1), shift.reshape(cout, 1))

    return out


# speedup vs baseline: 1.6400x; 1.6400x over previous
import jax
import jax.numpy as jnp
from jax.experimental import pallas as pl
from jax.experimental.pallas import tpu as pltpu

_BN_EPS = 1e-5
_VMEM_LIMIT = 64 * 1024 * 1024


def _conv_stats_kernel(x_ref, w_ref, ct_ref, s_ref, q_ref):
    kw_taps, kp, cout = w_ref.shape
    _, hp, wp, cin = x_ref.shape
    kh_taps = kw_taps
    h = hp - (kh_taps - 1)
    w = wp - (kw_taps - 1)

    xb = x_ref[0]
    xc = jnp.concatenate(
        [xb[kh:kh + h] for kh in range(kh_taps)]
        + [jnp.zeros((h, wp, kp - kh_taps * cin), jnp.bfloat16)], axis=-1)

    big = jnp.concatenate(
        [xc[:, kw:kw + w, :].reshape(h * w, kp) for kw in range(kw_taps)],
        axis=-1)
    acc = jax.lax.dot_general(
        big, w_ref[...].reshape(kw_taps * kp, cout),
        dimension_numbers=(((1,), (0,)), ((), ())),
        preferred_element_type=jnp.float32)

    s_ref[...] = jnp.sum(acc, axis=0).reshape(1, 1, cout)
    q_ref[...] = jnp.sum(acc * acc, axis=0).reshape(1, 1, cout)
    ct_ref[...] = acc.T.astype(jnp.bfloat16).reshape(1, cout, h * w)


def _bn_relu_kernel(ct_ref, sc_ref, sh_ref, o_ref):
    v = ct_ref[0].astype(jnp.float32)
    y = jnp.maximum(v * sc_ref[...] + sh_ref[...], 0.0)
    o_ref[...] = y.reshape(o_ref.shape)


def kernel(x_nchw, w_oihw, bias, gamma, beta):
    del bias
    n, cin, h, w = x_nchw.shape
    cout, _, kh, kw = w_oihw.shape
    pad = kh // 2
    hp, wp = h + 2 * pad, w + 2 * pad

    x = jnp.transpose(x_nchw, (0, 2, 3, 1)).astype(jnp.bfloat16)
    x = jnp.pad(x, ((0, 0), (pad, pad), (pad, pad), (0, 0)))

    kp = 4 * cin
    w3 = jnp.transpose(w_oihw, (3, 2, 1, 0)).reshape(kw, kh * cin, cout)
    w3 = jnp.pad(w3, ((0, 0), (0, kp - kh * cin), (0, 0))).astype(jnp.bfloat16)

    hw = h * w
    conv_flops = 2 * n * hw * kh * kw * cin * cout
    conv_bytes = 2 * (x.size + w3.size + n * hw * cout)

    convt, csum, csq = pl.pallas_call(
        _conv_stats_kernel,
        grid=(n,),
        in_specs=[
            pl.BlockSpec((1, hp, wp, cin), lambda i: (i, 0, 0, 0)),
            pl.BlockSpec((kw, kp, cout), lambda i: (0, 0, 0)),
        ],
        out_specs=[
            pl.BlockSpec((1, cout, hw), lambda i: (i, 0, 0)),
            pl.BlockSpec((1, 1, cout), lambda i: (i, 0, 0)),
            pl.BlockSpec((1, 1, cout), lambda i: (i, 0, 0)),
        ],
        out_shape=(
            jax.ShapeDtypeStruct((n, cout, hw), jnp.bfloat16),
            jax.ShapeDtypeStruct((n, 1, cout), jnp.float32),
            jax.ShapeDtypeStruct((n, 1, cout), jnp.float32),
        ),
        compiler_params=pltpu.CompilerParams(
            dimension_semantics=("parallel",),
            vmem_limit_bytes=_VMEM_LIMIT),
        cost_estimate=pl.CostEstimate(
            flops=conv_flops, transcendentals=0, bytes_accessed=conv_bytes),
    )(x, w3)

    cnt = float(n * hw)
    mean = csum.sum(axis=(0, 1)) / cnt
    var = jnp.maximum(csq.sum(axis=(0, 1)) / cnt - mean * mean, 0.0)
    scale = gamma.astype(jnp.float32) * jax.lax.rsqrt(var + _BN_EPS)
    shift = beta.astype(jnp.float32) - mean * scale

    out = pl.pallas_call(
        _bn_relu_kernel,
        grid=(n,),
        in_specs=[
            pl.BlockSpec((1, cout, hw), lambda i: (i, 0, 0)),
            pl.BlockSpec((cout, 1), lambda i: (0, 0)),
            pl.BlockSpec((cout, 1), lambda i: (0, 0)),
        ],
        out_specs=pl.BlockSpec((1, cout, hw), lambda i: (i, 0, 0)),
        out_shape=jax.ShapeDtypeStruct((n, cout, hw), jnp.float32),
        compiler_params=pltpu.CompilerParams(
            dimension_semantics=("parallel",),
            vmem_limit_bytes=_VMEM_LIMIT),
        cost_estimate=pl.CostEstimate(
            flops=3 * n * hw * cout, transcendentals=0,
            bytes_accessed=6 * n * hw * cout),
    )(convt, scale.reshape(cout, 1), shift.reshape(cout, 1))

    return out.reshape(n, cout, h, w)
